# baseline (speedup 1.0000x reference)
import functools

import jax
import jax.numpy as jnp
from jax import lax
from jax.experimental import pallas as pl
from jax.experimental.pallas import tpu as pltpu
from jax.experimental.pallas import tpu_sc as plsc

F32 = jnp.float32
_EMB = 64
_LANES = 16
_LA, _LB = 104, 96
_CH = 16
_NBIN = 1024

_SC_PARAMS = dict(
    compiler_params=pltpu.CompilerParams(use_tc_tiling_on_sc=False,
                                         needs_layout_passes=False))


def _counts_sc(shapes, colors):
    B, L = shapes.shape
    info = plsc.get_sparse_core_info()
    NC, NS = info.num_cores, info.num_subcores
    RPW = B // (NC * NS)
    NCHUNK = RPW // _CH

    mesh = plsc.VectorSubcoreMesh(core_axis_name="c", subcore_axis_name="s")

    @functools.partial(
        pl.kernel,
        out_type=(jax.ShapeDtypeStruct((B // 8, 8, 8, 128), F32),
                  jax.ShapeDtypeStruct((B // 8, 8, 8, 128), F32)),
        mesh=mesh,
        compiler_params=pltpu.CompilerParams(needs_layout_passes=False),
        scratch_types=[
            pltpu.VMEM((2, 2, _CH, L), jnp.int32),
            pltpu.VMEM((2, 2, 8, 8, 128), F32),
            pltpu.SemaphoreType.DMA((2,)),
            pltpu.SemaphoreType.DMA((2,)),
        ],
    )
    def k(shapes_h, colors_h, cs_h, cc_h, idx, cnts, isem, csem):
        wid = lax.axis_index("s") * NC + lax.axis_index("c")
        base = wid * RPW
        idx_hs = (shapes_h, colors_h)
        cnt_hs = (cs_h, cc_h)

        lane = lax.iota(jnp.int32, _LANES)
        zeros = jnp.zeros((_LANES,), F32)
        ones = zeros + 1.0
        tailm = lane >= 8

        def stage_chunk(c, slot):
            row0 = pl.multiple_of(base + c * _CH, 8)
            for t in range(2):
                pltpu.async_copy(idx_hs[t].at[pl.ds(row0, _CH)],
                                 idx.at[slot, t], isem.at[slot])

        def wait_chunk(slot):
            for t in range(2):
                pltpu.make_async_copy(idx_hs[t].at[pl.ds(0, _CH)],
                                      idx.at[slot, t], isem.at[slot]).wait()

        stage_chunk(0, 0)
        stage_chunk(1, 1)

        def row_body(r, carry):
            blk = r // 8
            bslot = lax.rem(blk, 2)
            chunk = r // _CH
            cslot = lax.rem(chunk, 2)
            rr = r - chunk * _CH
            s = r - blk * 8
            sv = lax.broadcast_in_dim(s, (_LANES,), ())

            @pl.when(rr == 0)
            def _():
                wait_chunk(cslot)

            @pl.when((s == 0) & (blk >= 2))
            def _():
                for t in range(2):
                    pltpu.make_async_copy(
                        cnts.at[bslot, t], cnt_hs[t].at[0],
                        csem.at[bslot]).wait()

            for t in range(2):
                cref = cnts.at[bslot, t]
                for j in range(8):
                    for kk in range(8):
                        cref[j, s, pl.ds(kk * _LANES, _LANES)] = zeros
                for kk in range(12):
                    iv = idx[cslot, t, rr, pl.ds(kk * _LANES, _LANES)]
                    plsc.addupdate_scatter(
                        cref, [iv >> 7, sv, iv & 127], ones)
                iv = idx[cslot, t, rr, pl.ds(L - _LANES, _LANES)]
                plsc.addupdate_scatter(
                    cref, [iv >> 7, sv, iv & 127], ones, mask=tailm)

            @pl.when(s == 7)
            def _():
                row8 = (base + r) // 8
                for t in range(2):
                    pltpu.async_copy(cnts.at[bslot, t], cnt_hs[t].at[row8],
                                     csem.at[bslot])

            @pl.when((rr == _CH - 1) & (chunk + 2 < NCHUNK))
            def _():
                stage_chunk(chunk + 2, cslot)

            return carry

        lax.fori_loop(0, RPW, row_body, 0)

        for s in range(2):
            for t in range(2):
                pltpu.make_async_copy(
                    cnts.at[s, t], cnt_hs[t].at[0], csem.at[s]).wait()

    return k(shapes, colors)


def _pool_sc(clusters, cluster_emb):
    B, L = clusters.shape
    assert L == _LA + _LB
    info = plsc.get_sparse_core_info()
    NC, NS = info.num_cores, info.num_subcores
    RPW = B // (NC * NS)
    NCHUNK = RPW // _CH

    mesh = plsc.VectorSubcoreMesh(core_axis_name="c", subcore_axis_name="s")

    @functools.partial(
        pl.kernel,
        out_type=jax.ShapeDtypeStruct((B, _EMB), F32),
        mesh=mesh,
        scratch_types=[
            pltpu.VMEM((2, _CH, _LA), jnp.int32),
            pltpu.VMEM((2, _CH, _LB), jnp.int32),
            pltpu.VMEM((2, L, _EMB), F32),
            pltpu.VMEM((RPW, _EMB), F32),
            pltpu.VMEM((_EMB,), F32),
            pltpu.SemaphoreType.DMA((2,)),
            pltpu.SemaphoreType.DMA((2,)),
        ],
        **_SC_PARAMS,
    )
    def k(clusters_h, ue_h, pool_h, idxA, idxB, buf, pooled, t0, isem, gsem):
        wid = lax.axis_index("s") * NC + lax.axis_index("c")
        base = wid * RPW

        pltpu.sync_copy(ue_h.at[0], t0)

        lane = lax.iota(jnp.int32, _LANES)
        zeros = jnp.zeros((_LANES,), F32)
        ones = zeros + 1.0
        tailf = jnp.where(lane >= 8, ones, zeros)

        def stage_chunk(c, slot):
            row0 = pl.multiple_of(base + c * _CH, 8)
            pltpu.async_copy(clusters_h.at[pl.ds(row0, _CH), pl.ds(0, _LA)],
                             idxA.at[slot], isem.at[slot])
            pltpu.async_copy(clusters_h.at[pl.ds(row0, _CH), pl.ds(_LA, _LB)],
                             idxB.at[slot], isem.at[slot])

        def wait_chunk(slot):
            pltpu.make_async_copy(
                clusters_h.at[pl.ds(0, _CH), pl.ds(0, _LA)],
                idxA.at[slot], isem.at[slot]).wait()
            pltpu.make_async_copy(
                clusters_h.at[pl.ds(0, _CH), pl.ds(_LA, _LB)],
                idxB.at[slot], isem.at[slot]).wait()

        def launch_gathers(r, gslot, cslot):
            rr = r - (r // _CH) * _CH
            pltpu.async_copy(ue_h.at[idxA.at[cslot, rr]],
                             buf.at[gslot, pl.ds(0, _LA)], gsem.at[gslot])
            pltpu.async_copy(ue_h.at[idxB.at[cslot, rr]],
                             buf.at[gslot, pl.ds(_LA, _LB)], gsem.at[gslot])

        def wait_gathers(gslot, cslot):
            pltpu.make_async_copy(ue_h.at[idxA.at[cslot, 0]],
                                  buf.at[gslot, pl.ds(0, _LA)],
                                  gsem.at[gslot]).wait()
            pltpu.make_async_copy(ue_h.at[idxB.at[cslot, 0]],
                                  buf.at[gslot, pl.ds(_LA, _LB)],
                                  gsem.at[gslot]).wait()

        stage_chunk(0, 0)
        stage_chunk(1, 1)
        wait_chunk(0)
        launch_gathers(0, 0, 0)

        def row_body(r, carry):
            gslot = lax.rem(r, 2)
            chunk = r // _CH
            cslot = lax.rem(chunk, 2)
            rr = r - chunk * _CH
            nxt = r + 1

            @pl.when(nxt < RPW)
            def _():
                nchunk = nxt // _CH
                ncslot = lax.rem(nchunk, 2)

                @pl.when(nxt == nchunk * _CH)
                def _():
                    wait_chunk(ncslot)

                launch_gathers(nxt, lax.rem(nxt, 2), ncslot)

            wait_gathers(gslot, cslot)

            def tok_body(l, accs):
                a = list(accs)
                for u in range(4):
                    for c in range(4):
                        a[c] = a[c] + buf[gslot, 4 * l + u,
                                          pl.ds(c * _LANES, _LANES)]
                return tuple(a)

            accs = lax.fori_loop(
                0, L // 4, tok_body, tuple(zeros for _ in range(4)))

            cnt = zeros
            for kk in range(6):
                iv = idxA[cslot, rr, pl.ds(kk * _LANES, _LANES)]
                cnt = cnt + jnp.where(iv == 0, ones, zeros)
            iv = idxA[cslot, rr, pl.ds(_LA - _LANES, _LANES)]
            cnt = cnt + jnp.where(iv == 0, tailf, zeros)
            for kk in range(6):
                iv = idxB[cslot, rr, pl.ds(kk * _LANES, _LANES)]
                cnt = cnt + jnp.where(iv == 0, ones, zeros)
            n0 = lax.broadcast_in_dim(jnp.sum(cnt), (_LANES,), ())
            for c in range(4):
                pooled[r, pl.ds(c * _LANES, _LANES)] = (
                    accs[c] - n0 * t0[pl.ds(c * _LANES, _LANES)])

            @pl.when((rr == _CH - 1) & (chunk + 2 < NCHUNK))
            def _():
                stage_chunk(chunk + 2, cslot)

            return carry

        lax.fori_loop(0, RPW, row_body, 0)
        pltpu.sync_copy(pooled, pool_h.at[pl.ds(pl.multiple_of(base, 8), RPW)])

    return k(clusters, cluster_emb)


def _mlp_tc(pool_u, cs4, cc4, mask, se_pad, ce_pad, W1, b1, W2, b2):
    B = pool_u.shape[0]
    NCLS = W2.shape[1]

    def k(pu_ref, cs_ref, cc_ref, mask_ref, se_ref, ce_ref,
          w1_ref, b1_ref, w2_ref, b2_ref, out_ref):
        ps = jnp.zeros((B, _EMB), F32)
        pc = jnp.zeros((B, _EMB), F32)
        cs = cs_ref[...]
        cc = cc_ref[...]
        se = se_ref[...]
        ce = ce_ref[...]
        for j in range(_NBIN // 128):
            tabs = se[j * 128:(j + 1) * 128, :]
            tabc = ce[j * 128:(j + 1) * 128, :]
            ps = ps + jnp.dot(cs[:, j].reshape(B, 128), tabs,
                              preferred_element_type=F32)
            pc = pc + jnp.dot(cc[:, j].reshape(B, 128), tabc,
                              preferred_element_type=F32)
        pooled = jnp.concatenate([ps, pc, pu_ref[...]], axis=1)
        ms = jnp.sum(mask_ref[...], axis=1, keepdims=True)
        h = jnp.dot(pooled / ms, w1_ref[...], preferred_element_type=F32)
        h = jnp.maximum(h + b1_ref[...], 0.0)
        out_ref[...] = jnp.dot(h, w2_ref[...],
                               preferred_element_type=F32) + b2_ref[...]

    return pl.pallas_call(
        k,
        out_shape=jax.ShapeDtypeStruct((B, NCLS), F32),
    )(pool_u, cs4, cc4, mask, se_pad, ce_pad,
      W1, b1.reshape(1, -1), W2, b2.reshape(1, -1))


def kernel(shapes, colors, clusters, mask, shape_emb, color_emb, cluster_emb,
           W1, b1, W2, b2):
    cs4, cc4 = _counts_sc(shapes, colors)
    pool_u = _pool_sc(clusters, cluster_emb)
    npad = _NBIN - shape_emb.shape[0]
    se_pad = jnp.pad(shape_emb.at[0].set(0.0), ((0, npad), (0, 0)))
    ce_pad = jnp.pad(color_emb.at[0].set(0.0), ((0, npad), (0, 0)))
    return _mlp_tc(pool_u, cs4, cc4, mask, se_pad, ce_pad, W1, b1, W2, b2)


# speedup vs baseline: 1.1026x; 1.1026x over previous
import functools

import jax
import jax.numpy as jnp
from jax import lax
from jax.experimental import pallas as pl
from jax.experimental.pallas import tpu as pltpu
from jax.experimental.pallas import tpu_sc as plsc

F32 = jnp.float32
_EMB = 64
_LANES = 16
_LA, _LB = 104, 96
_CH = 16
_NBIN = 1024

_SC_PARAMS = dict(
    compiler_params=pltpu.CompilerParams(use_tc_tiling_on_sc=False,
                                         needs_layout_passes=False))


def _counts_sc(shapes, colors):
    B, L = shapes.shape
    info = plsc.get_sparse_core_info()
    NC, NS = info.num_cores, info.num_subcores
    RPW = B // (NC * NS)
    NCHUNK = RPW // _CH

    mesh = plsc.VectorSubcoreMesh(core_axis_name="c", subcore_axis_name="s")

    @functools.partial(
        pl.kernel,
        out_type=(jax.ShapeDtypeStruct((B // 8, 8, 8, 128), F32),
                  jax.ShapeDtypeStruct((B // 8, 8, 8, 128), F32)),
        mesh=mesh,
        compiler_params=pltpu.CompilerParams(needs_layout_passes=False),
        scratch_types=[
            pltpu.VMEM((2, 2, _CH, L), jnp.int32),
            pltpu.VMEM((2, 2, 8, 8, 128), F32),
            pltpu.SemaphoreType.DMA((2,)),
            pltpu.SemaphoreType.DMA((2,)),
        ],
    )
    def k(shapes_h, colors_h, cs_h, cc_h, idx, cnts, isem, csem):
        wid = lax.axis_index("s") * NC + lax.axis_index("c")
        base = wid * RPW
        idx_hs = (shapes_h, colors_h)
        cnt_hs = (cs_h, cc_h)

        lane = lax.iota(jnp.int32, _LANES)
        zeros = jnp.zeros((_LANES,), F32)
        ones = zeros + 1.0
        tailm = lane >= 8

        def stage_chunk(c, slot):
            row0 = pl.multiple_of(base + c * _CH, 8)
            for t in range(2):
                pltpu.async_copy(idx_hs[t].at[pl.ds(row0, _CH)],
                                 idx.at[slot, t], isem.at[slot])

        def wait_chunk(slot):
            for t in range(2):
                pltpu.make_async_copy(idx_hs[t].at[pl.ds(0, _CH)],
                                      idx.at[slot, t], isem.at[slot]).wait()

        stage_chunk(0, 0)
        stage_chunk(1, 1)

        def row_body(r, carry):
            blk = r // 8
            bslot = lax.rem(blk, 2)
            chunk = r // _CH
            cslot = lax.rem(chunk, 2)
            rr = r - chunk * _CH
            s = r - blk * 8
            sv = lax.broadcast_in_dim(s, (_LANES,), ())

            @pl.when(rr == 0)
            def _():
                wait_chunk(cslot)

            @pl.when((s == 0) & (blk >= 2))
            def _():
                for t in range(2):
                    pltpu.make_async_copy(
                        cnts.at[bslot, t], cnt_hs[t].at[0],
                        csem.at[bslot]).wait()

            for t in range(2):
                cref = cnts.at[bslot, t]
                for j in range(8):
                    for kk in range(8):
                        cref[j, s, pl.ds(kk * _LANES, _LANES)] = zeros
                for kk in range(12):
                    iv = idx[cslot, t, rr, pl.ds(kk * _LANES, _LANES)]
                    plsc.addupdate_scatter(
                        cref, [iv >> 7, sv, iv & 127], ones)
                iv = idx[cslot, t, rr, pl.ds(L - _LANES, _LANES)]
                plsc.addupdate_scatter(
                    cref, [iv >> 7, sv, iv & 127], ones, mask=tailm)

            @pl.when(s == 7)
            def _():
                row8 = (base + r) // 8
                for t in range(2):
                    pltpu.async_copy(cnts.at[bslot, t], cnt_hs[t].at[row8],
                                     csem.at[bslot])

            @pl.when((rr == _CH - 1) & (chunk + 2 < NCHUNK))
            def _():
                stage_chunk(chunk + 2, cslot)

            return carry

        lax.fori_loop(0, RPW, row_body, 0)

        for s in range(2):
            for t in range(2):
                pltpu.make_async_copy(
                    cnts.at[s, t], cnt_hs[t].at[0], csem.at[s]).wait()

    return k(shapes, colors)


def _pool_sc(clusters, cluster_emb):
    B, L = clusters.shape
    assert L == _LA + _LB
    info = plsc.get_sparse_core_info()
    NC, NS = info.num_cores, info.num_subcores
    RPW = B // (NC * NS)
    NCHUNK = RPW // _CH

    mesh = plsc.VectorSubcoreMesh(core_axis_name="c", subcore_axis_name="s")

    @functools.partial(
        pl.kernel,
        out_type=jax.ShapeDtypeStruct((B, _EMB), F32),
        mesh=mesh,
        scratch_types=[
            pltpu.VMEM((2, _CH, _LA), jnp.int32),
            pltpu.VMEM((2, _CH, _LB), jnp.int32),
            pltpu.VMEM((2, L, _EMB), F32),
            pltpu.VMEM((RPW, _EMB), F32),
            pltpu.VMEM((_EMB,), F32),
            pltpu.SemaphoreType.DMA((2,)),
            pltpu.SemaphoreType.DMA((2,)),
        ],
        **_SC_PARAMS,
    )
    def k(clusters_h, ue_h, pool_h, idxA, idxB, buf, pooled, t0, isem, gsem):
        wid = lax.axis_index("s") * NC + lax.axis_index("c")
        base = wid * RPW

        pltpu.sync_copy(ue_h.at[0], t0)

        lane = lax.iota(jnp.int32, _LANES)
        zeros = jnp.zeros((_LANES,), F32)
        ones = zeros + 1.0
        tailf = jnp.where(lane >= 8, ones, zeros)

        def stage_chunk(c, slot):
            row0 = pl.multiple_of(base + c * _CH, 8)
            pltpu.async_copy(clusters_h.at[pl.ds(row0, _CH), pl.ds(0, _LA)],
                             idxA.at[slot], isem.at[slot])
            pltpu.async_copy(clusters_h.at[pl.ds(row0, _CH), pl.ds(_LA, _LB)],
                             idxB.at[slot], isem.at[slot])

        def wait_chunk(slot):
            pltpu.make_async_copy(
                clusters_h.at[pl.ds(0, _CH), pl.ds(0, _LA)],
                idxA.at[slot], isem.at[slot]).wait()
            pltpu.make_async_copy(
                clusters_h.at[pl.ds(0, _CH), pl.ds(_LA, _LB)],
                idxB.at[slot], isem.at[slot]).wait()

        def launch_gathers(r, gslot, cslot):
            rr = r - (r // _CH) * _CH
            pltpu.async_copy(ue_h.at[idxA.at[cslot, rr]],
                             buf.at[gslot, pl.ds(0, _LA)], gsem.at[gslot])
            pltpu.async_copy(ue_h.at[idxB.at[cslot, rr]],
                             buf.at[gslot, pl.ds(_LA, _LB)], gsem.at[gslot])

        def wait_gathers(gslot, cslot):
            pltpu.make_async_copy(ue_h.at[idxA.at[cslot, 0]],
                                  buf.at[gslot, pl.ds(0, _LA)],
                                  gsem.at[gslot]).wait()
            pltpu.make_async_copy(ue_h.at[idxB.at[cslot, 0]],
                                  buf.at[gslot, pl.ds(_LA, _LB)],
                                  gsem.at[gslot]).wait()

        stage_chunk(0, 0)
        stage_chunk(1, 1)
        wait_chunk(0)
        launch_gathers(0, 0, 0)

        def row_body(r, carry):
            gslot = lax.rem(r, 2)
            chunk = r // _CH
            cslot = lax.rem(chunk, 2)
            rr = r - chunk * _CH
            nxt = r + 1

            @pl.when(nxt < RPW)
            def _():
                nchunk = nxt // _CH
                ncslot = lax.rem(nchunk, 2)

                @pl.when(nxt == nchunk * _CH)
                def _():
                    wait_chunk(ncslot)

                launch_gathers(nxt, lax.rem(nxt, 2), ncslot)

            wait_gathers(gslot, cslot)

            def tok_body(l, accs):
                a = list(accs)
                for u in range(4):
                    for c in range(4):
                        a[c] = a[c] + buf[gslot, 4 * l + u,
                                          pl.ds(c * _LANES, _LANES)]
                return tuple(a)

            accs = lax.fori_loop(
                0, L // 4, tok_body, tuple(zeros for _ in range(4)))

            cnt = zeros
            for kk in range(6):
                iv = idxA[cslot, rr, pl.ds(kk * _LANES, _LANES)]
                cnt = cnt + jnp.where(iv == 0, ones, zeros)
            iv = idxA[cslot, rr, pl.ds(_LA - _LANES, _LANES)]
            cnt = cnt + jnp.where(iv == 0, tailf, zeros)
            for kk in range(6):
                iv = idxB[cslot, rr, pl.ds(kk * _LANES, _LANES)]
                cnt = cnt + jnp.where(iv == 0, ones, zeros)
            n0 = lax.broadcast_in_dim(jnp.sum(cnt), (_LANES,), ())
            for c in range(4):
                pooled[r, pl.ds(c * _LANES, _LANES)] = (
                    accs[c] - n0 * t0[pl.ds(c * _LANES, _LANES)])

            @pl.when((rr == _CH - 1) & (chunk + 2 < NCHUNK))
            def _():
                stage_chunk(chunk + 2, cslot)

            return carry

        lax.fori_loop(0, RPW, row_body, 0)
        pltpu.sync_copy(pooled, pool_h.at[pl.ds(pl.multiple_of(base, 8), RPW)])

    return k(clusters, cluster_emb)


def _mlp_tc(pool_u, cs4, cc4, mask, se_pad, ce_pad, W1, b1, W2, b2):
    B = pool_u.shape[0]
    NCLS = W2.shape[1]

    def k(pu_ref, cs_ref, cc_ref, mask_ref, se_ref, ce_ref,
          w1_ref, b1_ref, w2_ref, b2_ref, out_ref):
        ps = jnp.zeros((B, _EMB), F32)
        pc = jnp.zeros((B, _EMB), F32)
        cs = cs_ref[...]
        cc = cc_ref[...]
        se = se_ref[...]
        ce = ce_ref[...]
        for j in range(_NBIN // 128):
            tabs = se[j * 128:(j + 1) * 128, :]
            tabc = ce[j * 128:(j + 1) * 128, :]
            ps = ps + jnp.dot(cs[:, j].reshape(B, 128), tabs,
                              preferred_element_type=F32)
            pc = pc + jnp.dot(cc[:, j].reshape(B, 128), tabc,
                              preferred_element_type=F32)
        pooled = jnp.concatenate([ps, pc, pu_ref[...]], axis=1)
        ms = jnp.sum(mask_ref[...], axis=1, keepdims=True)
        h = jnp.dot(pooled / ms, w1_ref[...], preferred_element_type=F32)
        h = jnp.maximum(h + b1_ref[...], 0.0)
        out_ref[...] = jnp.dot(h, w2_ref[...],
                               preferred_element_type=F32) + b2_ref[...]

    return pl.pallas_call(
        k,
        out_shape=jax.ShapeDtypeStruct((B, NCLS), F32),
    )(pool_u, cs4, cc4, mask, se_pad, ce_pad,
      W1, b1.reshape(1, -1), W2, b2.reshape(1, -1))


def kernel(shapes, colors, clusters, mask, shape_emb, color_emb, cluster_emb,
           W1, b1, W2, b2):
    cs4, cc4 = _counts_sc(shapes, colors)
    clusters, cs4, cc4 = lax.optimization_barrier((clusters, cs4, cc4))
    pool_u = _pool_sc(clusters, cluster_emb)
    npad = _NBIN - shape_emb.shape[0]
    se_pad = jnp.pad(shape_emb.at[0].set(0.0), ((0, npad), (0, 0)))
    ce_pad = jnp.pad(color_emb.at[0].set(0.0), ((0, npad), (0, 0)))
    return _mlp_tc(pool_u, cs4, cc4, mask, se_pad, ce_pad, W1, b1, W2, b2)


# speedup vs baseline: 1.1378x; 1.0319x over previous
import functools

import jax
import jax.numpy as jnp
from jax import lax
from jax.experimental import pallas as pl
from jax.experimental.pallas import tpu as pltpu
from jax.experimental.pallas import tpu_sc as plsc

F32 = jnp.float32
_EMB = 64
_LANES = 16
_LA, _LB = 104, 96
_CH = 16
_NBIN = 1024

_SC_PARAMS = dict(
    compiler_params=pltpu.CompilerParams(use_tc_tiling_on_sc=False,
                                         needs_layout_passes=False))


def _counts_sc(shapes, colors):
    B, L = shapes.shape
    info = plsc.get_sparse_core_info()
    NC, NS = info.num_cores, info.num_subcores
    RPW = B // (NC * NS)
    NCHUNK = RPW // _CH

    mesh = plsc.VectorSubcoreMesh(core_axis_name="c", subcore_axis_name="s")

    @functools.partial(
        pl.kernel,
        out_type=(jax.ShapeDtypeStruct((B // 8, 8, 8, 128), F32),
                  jax.ShapeDtypeStruct((B // 8, 8, 8, 128), F32)),
        mesh=mesh,
        compiler_params=pltpu.CompilerParams(needs_layout_passes=False),
        scratch_types=[
            pltpu.VMEM((2, 2, _CH, L), jnp.int32),
            pltpu.VMEM((2, 2, 8, 8, 128), F32),
            pltpu.SemaphoreType.DMA((2,)),
            pltpu.SemaphoreType.DMA((2,)),
        ],
    )
    def k(shapes_h, colors_h, cs_h, cc_h, idx, cnts, isem, csem):
        wid = lax.axis_index("s") * NC + lax.axis_index("c")
        base = wid * RPW
        idx_hs = (shapes_h, colors_h)
        cnt_hs = (cs_h, cc_h)

        lane = lax.iota(jnp.int32, _LANES)
        zeros = jnp.zeros((_LANES,), F32)
        ones = zeros + 1.0
        tailm = lane >= 8

        def stage_chunk(c, slot):
            row0 = pl.multiple_of(base + c * _CH, 8)
            for t in range(2):
                pltpu.async_copy(idx_hs[t].at[pl.ds(row0, _CH)],
                                 idx.at[slot, t], isem.at[slot])

        def wait_chunk(slot):
            for t in range(2):
                pltpu.make_async_copy(idx_hs[t].at[pl.ds(0, _CH)],
                                      idx.at[slot, t], isem.at[slot]).wait()

        stage_chunk(0, 0)
        stage_chunk(1, 1)

        def row_body(r, carry):
            blk = r // 8
            bslot = lax.rem(blk, 2)
            chunk = r // _CH
            cslot = lax.rem(chunk, 2)
            rr = r - chunk * _CH
            s = r - blk * 8
            sv = lax.broadcast_in_dim(s, (_LANES,), ())

            @pl.when(rr == 0)
            def _():
                wait_chunk(cslot)

            @pl.when((s == 0) & (blk >= 2))
            def _():
                for t in range(2):
                    pltpu.make_async_copy(
                        cnts.at[bslot, t], cnt_hs[t].at[0],
                        csem.at[bslot]).wait()

            for t in range(2):
                cref = cnts.at[bslot, t]
                for j in range(8):
                    for kk in range(8):
                        cref[j, s, pl.ds(kk * _LANES, _LANES)] = zeros
                for kk in range(12):
                    iv = idx[cslot, t, rr, pl.ds(kk * _LANES, _LANES)]
                    plsc.addupdate_scatter(
                        cref, [iv >> 7, sv, iv & 127], ones)
                iv = idx[cslot, t, rr, pl.ds(L - _LANES, _LANES)]
                plsc.addupdate_scatter(
                    cref, [iv >> 7, sv, iv & 127], ones, mask=tailm)

            @pl.when(s == 7)
            def _():
                row8 = (base + r) // 8
                for t in range(2):
                    pltpu.async_copy(cnts.at[bslot, t], cnt_hs[t].at[row8],
                                     csem.at[bslot])

            @pl.when((rr == _CH - 1) & (chunk + 2 < NCHUNK))
            def _():
                stage_chunk(chunk + 2, cslot)

            return carry

        lax.fori_loop(0, RPW, row_body, 0)

        for s in range(2):
            for t in range(2):
                pltpu.make_async_copy(
                    cnts.at[s, t], cnt_hs[t].at[0], csem.at[s]).wait()

    return k(shapes, colors)


def _pool_sc(clusters, cluster_emb, dep):
    B, L = clusters.shape
    assert L == _LA + _LB
    info = plsc.get_sparse_core_info()
    NC, NS = info.num_cores, info.num_subcores
    RPW = B // (NC * NS)
    NCHUNK = RPW // _CH

    mesh = plsc.VectorSubcoreMesh(core_axis_name="c", subcore_axis_name="s")

    @functools.partial(
        pl.kernel,
        out_type=jax.ShapeDtypeStruct((B, _EMB), F32),
        mesh=mesh,
        scratch_types=[
            pltpu.VMEM((2, _CH, _LA), jnp.int32),
            pltpu.VMEM((2, _CH, _LB), jnp.int32),
            pltpu.VMEM((2, L, _EMB), F32),
            pltpu.VMEM((RPW, _EMB), F32),
            pltpu.VMEM((_EMB,), F32),
            pltpu.SemaphoreType.DMA((2,)),
            pltpu.SemaphoreType.DMA((2,)),
        ],
        **_SC_PARAMS,
    )
    def k(clusters_h, ue_h, dep_h, pool_h, idxA, idxB, buf, pooled, t0,
          isem, gsem):
        wid = lax.axis_index("s") * NC + lax.axis_index("c")
        base = wid * RPW

        pltpu.sync_copy(ue_h.at[0], t0)

        lane = lax.iota(jnp.int32, _LANES)
        zeros = jnp.zeros((_LANES,), F32)
        ones = zeros + 1.0
        tailf = jnp.where(lane >= 8, ones, zeros)

        def stage_chunk(c, slot):
            row0 = pl.multiple_of(base + c * _CH, 8)
            pltpu.async_copy(clusters_h.at[pl.ds(row0, _CH), pl.ds(0, _LA)],
                             idxA.at[slot], isem.at[slot])
            pltpu.async_copy(clusters_h.at[pl.ds(row0, _CH), pl.ds(_LA, _LB)],
                             idxB.at[slot], isem.at[slot])

        def wait_chunk(slot):
            pltpu.make_async_copy(
                clusters_h.at[pl.ds(0, _CH), pl.ds(0, _LA)],
                idxA.at[slot], isem.at[slot]).wait()
            pltpu.make_async_copy(
                clusters_h.at[pl.ds(0, _CH), pl.ds(_LA, _LB)],
                idxB.at[slot], isem.at[slot]).wait()

        def launch_gathers(r, gslot, cslot):
            rr = r - (r // _CH) * _CH
            pltpu.async_copy(ue_h.at[idxA.at[cslot, rr]],
                             buf.at[gslot, pl.ds(0, _LA)], gsem.at[gslot])
            pltpu.async_copy(ue_h.at[idxB.at[cslot, rr]],
                             buf.at[gslot, pl.ds(_LA, _LB)], gsem.at[gslot])

        def wait_gathers(gslot, cslot):
            pltpu.make_async_copy(ue_h.at[idxA.at[cslot, 0]],
                                  buf.at[gslot, pl.ds(0, _LA)],
                                  gsem.at[gslot]).wait()
            pltpu.make_async_copy(ue_h.at[idxB.at[cslot, 0]],
                                  buf.at[gslot, pl.ds(_LA, _LB)],
                                  gsem.at[gslot]).wait()

        stage_chunk(0, 0)
        stage_chunk(1, 1)
        wait_chunk(0)
        launch_gathers(0, 0, 0)

        def row_body(r, carry):
            gslot = lax.rem(r, 2)
            chunk = r // _CH
            cslot = lax.rem(chunk, 2)
            rr = r - chunk * _CH
            nxt = r + 1

            @pl.when(nxt < RPW)
            def _():
                nchunk = nxt // _CH
                ncslot = lax.rem(nchunk, 2)

                @pl.when(nxt == nchunk * _CH)
                def _():
                    wait_chunk(ncslot)

                launch_gathers(nxt, lax.rem(nxt, 2), ncslot)

            wait_gathers(gslot, cslot)

            def tok_body(l, accs):
                a = list(accs)
                for u in range(4):
                    for c in range(4):
                        a[c] = a[c] + buf[gslot, 4 * l + u,
                                          pl.ds(c * _LANES, _LANES)]
                return tuple(a)

            accs = lax.fori_loop(
                0, L // 4, tok_body, tuple(zeros for _ in range(4)))

            cnt = zeros
            for kk in range(6):
                iv = idxA[cslot, rr, pl.ds(kk * _LANES, _LANES)]
                cnt = cnt + jnp.where(iv == 0, ones, zeros)
            iv = idxA[cslot, rr, pl.ds(_LA - _LANES, _LANES)]
            cnt = cnt + jnp.where(iv == 0, tailf, zeros)
            for kk in range(6):
                iv = idxB[cslot, rr, pl.ds(kk * _LANES, _LANES)]
                cnt = cnt + jnp.where(iv == 0, ones, zeros)
            n0 = lax.broadcast_in_dim(jnp.sum(cnt), (_LANES,), ())
            for c in range(4):
                pooled[r, pl.ds(c * _LANES, _LANES)] = (
                    accs[c] - n0 * t0[pl.ds(c * _LANES, _LANES)])

            @pl.when((rr == _CH - 1) & (chunk + 2 < NCHUNK))
            def _():
                stage_chunk(chunk + 2, cslot)

            return carry

        lax.fori_loop(0, RPW, row_body, 0)
        pltpu.sync_copy(pooled, pool_h.at[pl.ds(pl.multiple_of(base, 8), RPW)])

    return k(clusters, cluster_emb, dep)


def _mlp_tc(pool_u, cs4, cc4, mask, se_pad, ce_pad, W1, b1, W2, b2):
    B = pool_u.shape[0]
    NCLS = W2.shape[1]

    def k(pu_ref, cs_ref, cc_ref, mask_ref, se_ref, ce_ref,
          w1_ref, b1_ref, w2_ref, b2_ref, out_ref):
        ps = jnp.zeros((B, _EMB), F32)
        pc = jnp.zeros((B, _EMB), F32)
        cs = cs_ref[...]
        cc = cc_ref[...]
        se = se_ref[...]
        ce = ce_ref[...]
        for j in range(_NBIN // 128):
            tabs = se[j * 128:(j + 1) * 128, :]
            tabc = ce[j * 128:(j + 1) * 128, :]
            ps = ps + jnp.dot(cs[:, j].reshape(B, 128), tabs,
                              preferred_element_type=F32)
            pc = pc + jnp.dot(cc[:, j].reshape(B, 128), tabc,
                              preferred_element_type=F32)
        pooled = jnp.concatenate([ps, pc, pu_ref[...]], axis=1)
        ms = jnp.sum(mask_ref[...], axis=1, keepdims=True)
        h = jnp.dot(pooled / ms, w1_ref[...], preferred_element_type=F32)
        h = jnp.maximum(h + b1_ref[...], 0.0)
        out_ref[...] = jnp.dot(h, w2_ref[...],
                               preferred_element_type=F32) + b2_ref[...]

    return pl.pallas_call(
        k,
        out_shape=jax.ShapeDtypeStruct((B, NCLS), F32),
    )(pool_u, cs4, cc4, mask, se_pad, ce_pad,
      W1, b1.reshape(1, -1), W2, b2.reshape(1, -1))


def kernel(shapes, colors, clusters, mask, shape_emb, color_emb, cluster_emb,
           W1, b1, W2, b2):
    cs4, cc4 = _counts_sc(shapes, colors)
    pool_u = _pool_sc(clusters, cluster_emb, cs4)
    npad = _NBIN - shape_emb.shape[0]
    se_pad = jnp.pad(shape_emb.at[0].set(0.0), ((0, npad), (0, 0)))
    ce_pad = jnp.pad(color_emb.at[0].set(0.0), ((0, npad), (0, 0)))
    return _mlp_tc(pool_u, cs4, cc4, mask, se_pad, ce_pad, W1, b1, W2, b2)


# speedup vs baseline: 1.1892x; 1.0451x over previous
import functools

import jax
import jax.numpy as jnp
from jax import lax
from jax.experimental import pallas as pl
from jax.experimental.pallas import tpu as pltpu
from jax.experimental.pallas import tpu_sc as plsc

F32 = jnp.float32
_EMB = 64
_LANES = 16
_LA, _LB = 104, 96
_CH = 16
_NBIN = 1024

_SC_PARAMS = dict(
    compiler_params=pltpu.CompilerParams(use_tc_tiling_on_sc=False,
                                         needs_layout_passes=False))


def _counts_sc(shapes, colors):
    B, L = shapes.shape
    info = plsc.get_sparse_core_info()
    NC, NS = info.num_cores, info.num_subcores
    RPW = B // (NC * NS)
    NCHUNK = RPW // _CH

    mesh = plsc.VectorSubcoreMesh(core_axis_name="c", subcore_axis_name="s")

    @functools.partial(
        pl.kernel,
        out_type=(jax.ShapeDtypeStruct((B // 8, 8, 8, 128), F32),
                  jax.ShapeDtypeStruct((B // 8, 8, 8, 128), F32)),
        mesh=mesh,
        compiler_params=pltpu.CompilerParams(needs_layout_passes=False),
        scratch_types=[
            pltpu.VMEM((2, 2, _CH, L), jnp.int32),
            pltpu.VMEM((2, 2, 8, 8, 128), F32),
            pltpu.SemaphoreType.DMA((2,)),
            pltpu.SemaphoreType.DMA((2,)),
        ],
    )
    def k(shapes_h, colors_h, cs_h, cc_h, idx, cnts, isem, csem):
        wid = lax.axis_index("s") * NC + lax.axis_index("c")
        base = wid * RPW
        idx_hs = (shapes_h, colors_h)
        cnt_hs = (cs_h, cc_h)

        lane = lax.iota(jnp.int32, _LANES)
        zeros = jnp.zeros((_LANES,), F32)
        ones = zeros + 1.0
        tailm = lane >= 8

        def stage_chunk(c, slot):
            row0 = pl.multiple_of(base + c * _CH, 8)
            for t in range(2):
                pltpu.async_copy(idx_hs[t].at[pl.ds(row0, _CH)],
                                 idx.at[slot, t], isem.at[slot])

        def wait_chunk(slot):
            for t in range(2):
                pltpu.make_async_copy(idx_hs[t].at[pl.ds(0, _CH)],
                                      idx.at[slot, t], isem.at[slot]).wait()

        stage_chunk(0, 0)
        stage_chunk(1, 1)

        def row_body(r, carry):
            blk = r // 8
            bslot = lax.rem(blk, 2)
            chunk = r // _CH
            cslot = lax.rem(chunk, 2)
            rr = r - chunk * _CH
            s = r - blk * 8
            sv = lax.broadcast_in_dim(s, (_LANES,), ())

            @pl.when(rr == 0)
            def _():
                wait_chunk(cslot)

            @pl.when((s == 0) & (blk >= 2))
            def _():
                for t in range(2):
                    pltpu.make_async_copy(
                        cnts.at[bslot, t], cnt_hs[t].at[0],
                        csem.at[bslot]).wait()

            for t in range(2):
                cref = cnts.at[bslot, t]
                for j in range(8):
                    for kk in range(8):
                        cref[j, s, pl.ds(kk * _LANES, _LANES)] = zeros
                for kk in range(12):
                    iv = idx[cslot, t, rr, pl.ds(kk * _LANES, _LANES)]
                    plsc.addupdate_scatter(
                        cref, [iv >> 7, sv, iv & 127], ones)
                iv = idx[cslot, t, rr, pl.ds(L - _LANES, _LANES)]
                plsc.addupdate_scatter(
                    cref, [iv >> 7, sv, iv & 127], ones, mask=tailm)

            @pl.when(s == 7)
            def _():
                row8 = (base + r) // 8
                for t in range(2):
                    pltpu.async_copy(cnts.at[bslot, t], cnt_hs[t].at[row8],
                                     csem.at[bslot])

            @pl.when((rr == _CH - 1) & (chunk + 2 < NCHUNK))
            def _():
                stage_chunk(chunk + 2, cslot)

            return carry

        lax.fori_loop(0, RPW, row_body, 0)

        for s in range(2):
            for t in range(2):
                pltpu.make_async_copy(
                    cnts.at[s, t], cnt_hs[t].at[0], csem.at[s]).wait()

    return k(shapes, colors)


def _pool_sc(clusters, cluster_emb, dep):
    B, L = clusters.shape
    assert L == _LA + _LB
    info = plsc.get_sparse_core_info()
    NC, NS = info.num_cores, info.num_subcores
    RPW = B // (NC * NS)
    NCHUNK = RPW // _CH

    mesh = plsc.VectorSubcoreMesh(core_axis_name="c", subcore_axis_name="s")

    @functools.partial(
        pl.kernel,
        out_type=jax.ShapeDtypeStruct((B, _EMB), F32),
        mesh=mesh,
        scratch_types=[
            pltpu.VMEM((2, _CH, _LA), jnp.int32),
            pltpu.VMEM((2, _CH, _LB), jnp.int32),
            pltpu.VMEM((2, L, _EMB), F32),
            pltpu.VMEM((RPW, _EMB), F32),
            pltpu.VMEM((_EMB,), F32),
            pltpu.SemaphoreType.DMA((2,)),
            pltpu.SemaphoreType.DMA((2,)),
        ],
        **_SC_PARAMS,
    )
    def k(clusters_h, ue_h, dep_h, pool_h, idxA, idxB, buf, pooled, t0,
          isem, gsem):
        wid = lax.axis_index("s") * NC + lax.axis_index("c")
        base = wid * RPW

        pltpu.sync_copy(ue_h.at[0], t0)

        lane = lax.iota(jnp.int32, _LANES)
        zeros = jnp.zeros((_LANES,), F32)
        ones = zeros + 1.0
        tailf = jnp.where(lane >= 8, ones, zeros)

        def stage_chunk(c, slot):
            row0 = pl.multiple_of(base + c * _CH, 8)
            pltpu.async_copy(clusters_h.at[pl.ds(row0, _CH), pl.ds(0, _LA)],
                             idxA.at[slot], isem.at[slot])
            pltpu.async_copy(clusters_h.at[pl.ds(row0, _CH), pl.ds(_LA, _LB)],
                             idxB.at[slot], isem.at[slot])

        def wait_chunk(slot):
            pltpu.make_async_copy(
                clusters_h.at[pl.ds(0, _CH), pl.ds(0, _LA)],
                idxA.at[slot], isem.at[slot]).wait()
            pltpu.make_async_copy(
                clusters_h.at[pl.ds(0, _CH), pl.ds(_LA, _LB)],
                idxB.at[slot], isem.at[slot]).wait()

        def launch_gathers(r, gslot, cslot):
            rr = r - (r // _CH) * _CH
            pltpu.async_copy(ue_h.at[idxA.at[cslot, rr]],
                             buf.at[gslot, pl.ds(0, _LA)], gsem.at[gslot])
            pltpu.async_copy(ue_h.at[idxB.at[cslot, rr]],
                             buf.at[gslot, pl.ds(_LA, _LB)], gsem.at[gslot])

        def wait_gathers(gslot, cslot):
            pltpu.make_async_copy(ue_h.at[idxA.at[cslot, 0]],
                                  buf.at[gslot, pl.ds(0, _LA)],
                                  gsem.at[gslot]).wait()
            pltpu.make_async_copy(ue_h.at[idxB.at[cslot, 0]],
                                  buf.at[gslot, pl.ds(_LA, _LB)],
                                  gsem.at[gslot]).wait()

        stage_chunk(0, 0)
        stage_chunk(1, 1)
        wait_chunk(0)
        launch_gathers(0, 0, 0)

        def row_body(r, carry):
            gslot = lax.rem(r, 2)
            chunk = r // _CH
            cslot = lax.rem(chunk, 2)
            rr = r - chunk * _CH
            nxt = r + 1

            @pl.when(nxt < RPW)
            def _():
                nchunk = nxt // _CH
                ncslot = lax.rem(nchunk, 2)

                @pl.when(nxt == nchunk * _CH)
                def _():
                    wait_chunk(ncslot)

                launch_gathers(nxt, lax.rem(nxt, 2), ncslot)

            wait_gathers(gslot, cslot)

            def tok_body(l, accs):
                a = list(accs)
                for u in range(4):
                    for c in range(4):
                        a[c] = a[c] + buf[gslot, 4 * l + u,
                                          pl.ds(c * _LANES, _LANES)]
                return tuple(a)

            accs = lax.fori_loop(
                0, L // 4, tok_body, tuple(zeros for _ in range(4)))

            cnt = zeros
            for kk in range(6):
                iv = idxA[cslot, rr, pl.ds(kk * _LANES, _LANES)]
                cnt = cnt + jnp.where(iv == 0, ones, zeros)
            iv = idxA[cslot, rr, pl.ds(_LA - _LANES, _LANES)]
            cnt = cnt + jnp.where(iv == 0, tailf, zeros)
            for kk in range(6):
                iv = idxB[cslot, rr, pl.ds(kk * _LANES, _LANES)]
                cnt = cnt + jnp.where(iv == 0, ones, zeros)
            n0 = lax.broadcast_in_dim(jnp.sum(cnt), (_LANES,), ())
            for c in range(4):
                pooled[r, pl.ds(c * _LANES, _LANES)] = (
                    accs[c] - n0 * t0[pl.ds(c * _LANES, _LANES)])

            @pl.when((rr == _CH - 1) & (chunk + 2 < NCHUNK))
            def _():
                stage_chunk(chunk + 2, cslot)

            return carry

        lax.fori_loop(0, RPW, row_body, 0)
        pltpu.sync_copy(pooled, pool_h.at[pl.ds(pl.multiple_of(base, 8), RPW)])

    return k(clusters, cluster_emb, dep)


def _cnt_matmul_tc(cs4, cc4, se_pad, ce_pad):
    B = cs4.shape[0] * 8

    def k(cs_ref, cc_ref, se_ref, ce_ref, out_ref):
        ps = jnp.zeros((B, _EMB), F32)
        pc = jnp.zeros((B, _EMB), F32)
        cs = cs_ref[...]
        cc = cc_ref[...]
        se = se_ref[...]
        ce = ce_ref[...]
        for j in range(_NBIN // 128):
            tabs = se[j * 128:(j + 1) * 128, :]
            tabc = ce[j * 128:(j + 1) * 128, :]
            ps = ps + jnp.dot(cs[:, j].reshape(B, 128), tabs,
                              preferred_element_type=F32)
            pc = pc + jnp.dot(cc[:, j].reshape(B, 128), tabc,
                              preferred_element_type=F32)
        out_ref[...] = jnp.concatenate([ps, pc], axis=1)

    return pl.pallas_call(
        k,
        out_shape=jax.ShapeDtypeStruct((B, 2 * _EMB), F32),
    )(cs4, cc4, se_pad, ce_pad)


def _mlp_tc(psc, pool_u, mask, W1, b1, W2, b2):
    B = pool_u.shape[0]
    NCLS = W2.shape[1]

    def k(psc_ref, pu_ref, mask_ref, w1_ref, b1_ref, w2_ref, b2_ref, out_ref):
        pooled = jnp.concatenate([psc_ref[...], pu_ref[...]], axis=1)
        ms = jnp.sum(mask_ref[...], axis=1, keepdims=True)
        h = jnp.dot(pooled / ms, w1_ref[...], preferred_element_type=F32)
        h = jnp.maximum(h + b1_ref[...], 0.0)
        out_ref[...] = jnp.dot(h, w2_ref[...],
                               preferred_element_type=F32) + b2_ref[...]

    return pl.pallas_call(
        k,
        out_shape=jax.ShapeDtypeStruct((B, NCLS), F32),
    )(psc, pool_u, mask, W1, b1.reshape(1, -1), W2, b2.reshape(1, -1))


def kernel(shapes, colors, clusters, mask, shape_emb, color_emb, cluster_emb,
           W1, b1, W2, b2):
    cs4, cc4 = _counts_sc(shapes, colors)
    pool_u = _pool_sc(clusters, cluster_emb, cs4)
    npad = _NBIN - shape_emb.shape[0]
    se_pad = jnp.pad(shape_emb.at[0].set(0.0), ((0, npad), (0, 0)))
    ce_pad = jnp.pad(color_emb.at[0].set(0.0), ((0, npad), (0, 0)))
    psc = _cnt_matmul_tc(cs4, cc4, se_pad, ce_pad)
    return _mlp_tc(psc, pool_u, mask, W1, b1, W2, b2)
